# Initial kernel scaffold; baseline (speedup 1.0000x reference)
import jax
import jax.numpy as jnp
from jax.experimental import pallas as pl


def kernel(position_ids, cos_emb, sin_emb):
    raise NotImplementedError("write your pallas kernel here")


# speedup vs baseline: 1.5029x; 1.5029x over previous
import functools

import jax
import jax.numpy as jnp
from jax import lax
from jax.experimental import pallas as pl
from jax.experimental.pallas import tpu as pltpu
from jax.experimental.pallas import tpu_sc as plsc

D = 128
C = 128

_info = plsc.get_sparse_core_info()
_NC, _NS = _info.num_cores, _info.num_subcores
NW = _NC * _NS

_mesh = plsc.VectorSubcoreMesh(core_axis_name="c", subcore_axis_name="s")


def _make_gather(n_total: int):
    assert n_total % (NW * C) == 0
    bpw = n_total // NW
    nch = bpw // C

    @functools.partial(
        pl.kernel,
        mesh=_mesh,
        out_type=[
            jax.ShapeDtypeStruct((n_total, D), jnp.float32),
            jax.ShapeDtypeStruct((n_total, D), jnp.float32),
        ],
        scratch_types=[
            pltpu.VMEM((nch, C), jnp.int32),
            pltpu.VMEM((C, D), jnp.float32),
            pltpu.VMEM((C, D), jnp.float32),
            pltpu.SemaphoreType.DMA,
        ],
    )
    def gather_kernel(idx_hbm, cos_hbm, sin_hbm, cos_out, sin_out,
                      idx_v, cos_buf, sin_buf, sem):
        wid = lax.axis_index("s") * _NC + lax.axis_index("c")
        base = wid * bpw
        pltpu.sync_copy(idx_hbm.at[wid], idx_v)
        for ch in range(nch):
            cp_cos = pltpu.async_copy(cos_hbm.at[idx_v.at[ch]], cos_buf, sem)
            cp_sin = pltpu.async_copy(sin_hbm.at[idx_v.at[ch]], sin_buf, sem)
            cp_cos.wait()
            cp_sin.wait()
            off = base + ch * C
            pltpu.sync_copy(cos_buf, cos_out.at[pl.ds(off, C)])
            pltpu.sync_copy(sin_buf, sin_out.at[pl.ds(off, C)])

    return gather_kernel


def kernel(position_ids, cos_emb, sin_emb):
    b, s = position_ids.shape
    n = b * s
    idx3 = position_ids.astype(jnp.int32).reshape(NW, n // (NW * C), C)
    g = _make_gather(n)
    cos_flat, sin_flat = g(idx3, cos_emb, sin_emb)
    return (cos_flat.reshape(b, s, D), sin_flat.reshape(b, s, D))


# speedup vs baseline: 1.6670x; 1.1092x over previous
import functools

import jax
import jax.numpy as jnp
from jax import lax
from jax.experimental import pallas as pl
from jax.experimental.pallas import tpu as pltpu
from jax.experimental.pallas import tpu_sc as plsc

D = 128
C = 128

_info = plsc.get_sparse_core_info()
_NC, _NS = _info.num_cores, _info.num_subcores
NW = _NC * _NS

_mesh = plsc.VectorSubcoreMesh(core_axis_name="c", subcore_axis_name="s")

NSLOT = 3


def _make_gather(n_total: int):
    assert n_total % (NW * C) == 0
    bpw = n_total // NW
    nch = bpw // C

    @functools.partial(
        pl.kernel,
        mesh=_mesh,
        out_type=[
            jax.ShapeDtypeStruct((n_total, D), jnp.float32),
            jax.ShapeDtypeStruct((n_total, D), jnp.float32),
        ],
        scratch_types=[
            pltpu.VMEM((nch, C), jnp.int32),
            pltpu.VMEM((NSLOT, C, D), jnp.float32),
            pltpu.VMEM((NSLOT, C, D), jnp.float32),
            pltpu.SemaphoreType.DMA,
            pltpu.SemaphoreType.DMA,
        ],
    )
    def gather_kernel(idx_hbm, cos_hbm, sin_hbm, cos_out, sin_out,
                      idx_v, cos_buf, sin_buf, gsem, wsem):
        wid = lax.axis_index("s") * _NC + lax.axis_index("c")
        base = wid * bpw
        pltpu.sync_copy(idx_hbm.at[wid], idx_v)

        def fire_gather(ch):
            slot = ch % NSLOT
            return (
                pltpu.async_copy(cos_hbm.at[idx_v.at[ch]], cos_buf.at[slot], gsem),
                pltpu.async_copy(sin_hbm.at[idx_v.at[ch]], sin_buf.at[slot], gsem),
            )

        def fire_write(ch):
            slot = ch % NSLOT
            dst = pl.ds(base + ch * C, C)
            return (
                pltpu.async_copy(cos_buf.at[slot], cos_out.at[dst], wsem),
                pltpu.async_copy(sin_buf.at[slot], sin_out.at[dst], wsem),
            )

        g = [None] * nch
        w = [None] * nch
        for ch in range(min(NSLOT - 1, nch)):
            g[ch] = fire_gather(ch)
        for ch in range(nch):
            nxt = ch + NSLOT - 1
            if nxt < nch:
                prev = nxt - NSLOT
                if prev >= 0:
                    w[prev][0].wait()
                    w[prev][1].wait()
                g[nxt] = fire_gather(nxt)
            g[ch][0].wait()
            g[ch][1].wait()
            w[ch] = fire_write(ch)
        for ch in range(max(0, nch - NSLOT), nch):
            if w[ch] is not None:
                w[ch][0].wait()
                w[ch][1].wait()

    return gather_kernel


def kernel(position_ids, cos_emb, sin_emb):
    b, s = position_ids.shape
    n = b * s
    idx3 = position_ids.astype(jnp.int32).reshape(NW, n // (NW * C), C)
    g = _make_gather(n)
    cos_flat, sin_flat = g(idx3, cos_emb, sin_emb)
    return (cos_flat.reshape(b, s, D), sin_flat.reshape(b, s, D))
